# baseline (speedup 1.0000x reference)
import functools

import jax
import jax.numpy as jnp
from jax import lax
from jax.experimental import pallas as pl
from jax.experimental.pallas import tpu as pltpu
from jax.experimental.pallas import tpu_sc as plsc

_GROW = 8
_NBUF = 4


@functools.cache
def _build(b, s, d):
    info = plsc.get_sparse_core_info()
    nw = info.num_cores * info.num_subcores
    rows_per_worker = b // nw
    groups = rows_per_worker // _GROW

    mesh = plsc.VectorSubcoreMesh(core_axis_name="c", subcore_axis_name="s")

    @functools.partial(
        pl.kernel,
        mesh=mesh,
        compiler_params=pltpu.CompilerParams(use_tc_tiling_on_sc=False),
        out_type=jax.ShapeDtypeStruct((b, s, d), jnp.float32),
        scratch_types=[
            pltpu.VMEM((rows_per_worker, s), jnp.int32),
            pltpu.VMEM((_NBUF, _GROW, s, d), jnp.float32),
            pltpu.SemaphoreType.DMA,
            pltpu.SemaphoreType.DMA,
            pltpu.SemaphoreType.DMA,
            pltpu.SemaphoreType.DMA,
            pltpu.SemaphoreType.DMA,
            pltpu.SemaphoreType.DMA,
            pltpu.SemaphoreType.DMA,
            pltpu.SemaphoreType.DMA,
        ],
    )
    def gather_kernel(idx_hbm, table_hbm, out_hbm, idx_v, rows_v,
                      gsem0, gsem1, gsem2, gsem3,
                      wsem0, wsem1, wsem2, wsem3):
        wid = lax.axis_index("s") * info.num_cores + lax.axis_index("c")
        row0 = wid * rows_per_worker
        gsems = (gsem0, gsem1, gsem2, gsem3)
        wsems = (wsem0, wsem1, wsem2, wsem3)

        pltpu.sync_copy(idx_hbm.at[pl.ds(row0, rows_per_worker)], idx_v)

        def fire_gather(bf, g):
            for r in range(_GROW):
                pltpu.async_copy(
                    table_hbm.at[idx_v.at[g * _GROW + r]],
                    rows_v.at[bf].at[r],
                    gsems[bf],
                )

        def wait_gather(bf, g):
            for r in range(_GROW):
                pltpu.make_async_copy(
                    table_hbm.at[idx_v.at[g * _GROW + r]],
                    rows_v.at[bf].at[r],
                    gsems[bf],
                ).wait()

        def fire_wb(bf, g):
            pltpu.async_copy(
                rows_v.at[bf],
                out_hbm.at[pl.ds(row0 + g * _GROW, _GROW)],
                wsems[bf],
            )

        def wait_wb(bf, g):
            pltpu.make_async_copy(
                rows_v.at[bf],
                out_hbm.at[pl.ds(row0 + g * _GROW, _GROW)],
                wsems[bf],
            ).wait()

        for g0 in range(_NBUF):
            fire_gather(g0, g0)

        def body(qp, carry):
            for bf in range(_NBUF):
                g = _NBUF * qp + bf
                wait_gather(bf, g)
                fire_wb(bf, g)
                @pl.when(jnp.logical_and(g >= 2, g + 2 < groups))
                def _():
                    nxt = (bf + 2) % _NBUF
                    wait_wb(nxt, g - 2)
                    fire_gather(nxt, g + 2)
            return carry

        lax.fori_loop(0, groups // _NBUF, body, 0)

        for g in range(groups - _NBUF, groups):
            wait_wb(g % _NBUF, g)

    return gather_kernel


def kernel(idx, codebook):
    b, s = idx.shape
    d = codebook.shape[1]
    return _build(b, s, d)(idx.astype(jnp.int32), codebook)


# speedup vs baseline: 1.0053x; 1.0053x over previous
import functools

import jax
import jax.numpy as jnp
from jax import lax
from jax.experimental import pallas as pl
from jax.experimental.pallas import tpu as pltpu
from jax.experimental.pallas import tpu_sc as plsc

_IW = 128
_GROW = 2
_NBUF = 4


@functools.cache
def _build(n, d):
    info = plsc.get_sparse_core_info()
    nw = info.num_cores * info.num_subcores
    nrows = n // _IW
    rows_per_worker = nrows // nw
    groups = rows_per_worker // _GROW

    mesh = plsc.VectorSubcoreMesh(core_axis_name="c", subcore_axis_name="s")

    @functools.partial(
        pl.kernel,
        mesh=mesh,
        compiler_params=pltpu.CompilerParams(use_tc_tiling_on_sc=False),
        out_type=jax.ShapeDtypeStruct((n, d), jnp.float32),
        scratch_types=[
            pltpu.VMEM((rows_per_worker, _IW), jnp.int32),
            pltpu.VMEM((_NBUF, _GROW * _IW, d), jnp.float32),
            pltpu.SemaphoreType.DMA,
            pltpu.SemaphoreType.DMA,
            pltpu.SemaphoreType.DMA,
            pltpu.SemaphoreType.DMA,
            pltpu.SemaphoreType.DMA,
            pltpu.SemaphoreType.DMA,
            pltpu.SemaphoreType.DMA,
            pltpu.SemaphoreType.DMA,
        ],
    )
    def gather_kernel(idx_hbm, table_hbm, out_hbm, idx_v, rows_v,
                      gsem0, gsem1, gsem2, gsem3,
                      wsem0, wsem1, wsem2, wsem3):
        wid = lax.axis_index("s") * info.num_cores + lax.axis_index("c")
        row0 = wid * rows_per_worker
        gsems = (gsem0, gsem1, gsem2, gsem3)
        wsems = (wsem0, wsem1, wsem2, wsem3)

        pltpu.sync_copy(idx_hbm.at[pl.ds(row0, rows_per_worker)], idx_v)

        def fire_gather(bf, g):
            for r in range(_GROW):
                pltpu.async_copy(
                    table_hbm.at[idx_v.at[g * _GROW + r]],
                    rows_v.at[bf].at[pl.ds(r * _IW, _IW)],
                    gsems[bf],
                )

        def wait_gather(bf, g):
            for r in range(_GROW):
                pltpu.make_async_copy(
                    table_hbm.at[idx_v.at[g * _GROW + r]],
                    rows_v.at[bf].at[pl.ds(r * _IW, _IW)],
                    gsems[bf],
                ).wait()

        def fire_wb(bf, g):
            pltpu.async_copy(
                rows_v.at[bf],
                out_hbm.at[pl.ds((row0 + g * _GROW) * _IW, _GROW * _IW)],
                wsems[bf],
            )

        def wait_wb(bf, g):
            pltpu.make_async_copy(
                rows_v.at[bf],
                out_hbm.at[pl.ds((row0 + g * _GROW) * _IW, _GROW * _IW)],
                wsems[bf],
            ).wait()

        for g0 in range(_NBUF):
            fire_gather(g0, g0)

        def body(qp, carry):
            for bf in range(_NBUF):
                g = _NBUF * qp + bf
                wait_gather(bf, g)
                fire_wb(bf, g)
                @pl.when(jnp.logical_and(g >= 2, g + 2 < groups))
                def _():
                    nxt = (bf + 2) % _NBUF
                    wait_wb(nxt, g - 2)
                    fire_gather(nxt, g + 2)
            return carry

        lax.fori_loop(0, groups // _NBUF, body, 0)

        for g in range(groups - _NBUF, groups):
            wait_wb(g % _NBUF, g)

    return gather_kernel


def kernel(idx, codebook):
    b, s = idx.shape
    d = codebook.shape[1]
    n = b * s
    idx2 = idx.astype(jnp.int32).reshape(n // _IW, _IW)
    out = _build(n, d)(idx2, codebook)
    return out.reshape(b, s, d)
